# Initial kernel scaffold; baseline (speedup 1.0000x reference)
import functools

import jax
import jax.numpy as jnp
from jax import lax
from jax.experimental import pallas as pl
from jax.experimental.pallas import tpu as pltpu
from jax.experimental.pallas import tpu_sc as plsc

NC = 2
NS = 16
NW = NC * NS
CHUNK = 128
ROW_BLOCK = 2000


def _scatter_add_sc(table, dst_idx, src_idx, acc_rows, width, chunks_per_worker):
  stripe = acc_rows // NS

  mesh = plsc.VectorSubcoreMesh(core_axis_name="c", subcore_axis_name="s")

  @functools.partial(
      pl.kernel,
      out_type=jax.ShapeDtypeStruct((NC, acc_rows, width), jnp.float32),
      mesh=mesh,
      scratch_types=[
          pltpu.VMEM((CHUNK,), jnp.int32),
          pltpu.VMEM((CHUNK,), jnp.int32),
          pltpu.VMEM((CHUNK, width), jnp.float32),
          pltpu.VMEM((acc_rows // NS, width), jnp.float32),
          pltpu.VMEM_SHARED((acc_rows, width), jnp.float32),
          pltpu.SemaphoreType.DMA,
      ],
  )
  def k(table_hbm, dst_hbm, src_hbm, out_hbm, idx_d, idx_s, rows, zeros_v, acc, sem):
    cid = lax.axis_index("c")
    sid = lax.axis_index("s")
    wid = sid * NC + cid

    def zrow(r, carry):
      for c in range(width // 16):
        zeros_v[r, pl.ds(c * 16, 16)] = jnp.zeros((16,), jnp.float32)
      return carry
    lax.fori_loop(0, stripe, zrow, 0)
    pltpu.sync_copy(zeros_v, acc.at[pl.ds(sid * stripe, stripe)])
    plsc.subcore_barrier()

    def step(t, carry):
      base = (wid * chunks_per_worker + t) * CHUNK
      pltpu.sync_copy(dst_hbm.at[pl.ds(base, CHUNK)], idx_d)
      pltpu.sync_copy(src_hbm.at[pl.ds(base, CHUNK)], idx_s)
      pltpu.async_copy(table_hbm.at[idx_s], rows, sem).wait()
      pltpu.sync_copy(rows, acc.at[idx_d], add=True)
      return carry
    lax.fori_loop(0, chunks_per_worker, step, 0)

    plsc.subcore_barrier()
    pltpu.sync_copy(acc.at[pl.ds(sid * stripe, stripe)],
                    out_hbm.at[cid].at[pl.ds(sid * stripe, stripe)])

  return k(table, dst_idx, src_idx)


def _mm1_kernel(x_ref, w0t_ref, w1t_ref, b0_ref, b1_ref, vw0_ref, table_ref):
  x = x_ref[...]
  vw0 = jnp.dot(x, w0t_ref[...], preferred_element_type=jnp.float32)
  vw1 = jnp.dot(x, w1t_ref[...], preferred_element_type=jnp.float32)
  vw0_ref[...] = vw0 + b0_ref[...]
  r = x.shape[0]
  table_ref[...] = jnp.concatenate(
      [vw1 + b1_ref[...], jnp.ones((r, 1), jnp.float32),
       jnp.zeros((r, 15), jnp.float32)], axis=1)


def _mid_kernel(vw0_ref, p_ref, w0t_ref, w1t_ref, b0_ref, b1_ref, aux_ref,
                table_ref):
  p = p_ref[...]
  s = p[0] + p[1]
  nbr = s[:, :64]
  cnt = s[:, 64:65]
  dinv = 1.0 / cnt
  h = jnp.maximum((vw0_ref[...] + nbr) * dinv, 0.0)
  hw0 = jnp.dot(h, w0t_ref[...], preferred_element_type=jnp.float32) + b0_ref[...]
  hw1 = jnp.dot(h, w1t_ref[...], preferred_element_type=jnp.float32) + b1_ref[...]
  r = h.shape[0]
  aux_ref[...] = jnp.concatenate(
      [hw0, dinv, jnp.zeros((r, 4), jnp.float32)], axis=1)
  table_ref[...] = jnp.concatenate(
      [hw1, jnp.zeros((r, 13), jnp.float32)], axis=1)


def _final_kernel(aux_ref, p_ref, out_ref):
  p = p_ref[...]
  s = p[0] + p[1]
  out_ref[...] = (aux_ref[:, :3] + s[:, :3]) * aux_ref[:, 3:4]


def kernel(features, w0_1, b0_1, w1_1, b1_1, w0_2, b0_2, w1_2, b1_2, edges):
  V = features.shape[0]
  E = edges.shape[0]
  E2 = 2 * E
  per_worker = -(-E2 // (NW * CHUNK))
  pad_len = per_worker * NW * CHUNK - E2
  acc_rows = -(-(V + 1) // (NS * CHUNK)) * NS * CHUNK

  e0 = edges[:, 0]
  e1 = edges[:, 1]
  dst_idx = jnp.concatenate([e0, e1, jnp.full((pad_len,), V, jnp.int32)])
  src_idx = jnp.concatenate([e1, e0, jnp.zeros((pad_len,), jnp.int32)])

  grid = V // ROW_BLOCK

  vw0, table1 = pl.pallas_call(
      _mm1_kernel,
      grid=(grid,),
      in_specs=[
          pl.BlockSpec((ROW_BLOCK, 128), lambda i: (i, 0)),
          pl.BlockSpec((128, 64), lambda i: (0, 0)),
          pl.BlockSpec((128, 64), lambda i: (0, 0)),
          pl.BlockSpec((1, 64), lambda i: (0, 0)),
          pl.BlockSpec((1, 64), lambda i: (0, 0)),
      ],
      out_specs=[
          pl.BlockSpec((ROW_BLOCK, 64), lambda i: (i, 0)),
          pl.BlockSpec((ROW_BLOCK, 80), lambda i: (i, 0)),
      ],
      out_shape=[
          jax.ShapeDtypeStruct((V, 64), jnp.float32),
          jax.ShapeDtypeStruct((V, 80), jnp.float32),
      ],
  )(features, w0_1.T, w1_1.T, b0_1[None, :], b1_1[None, :])

  p1 = _scatter_add_sc(table1, dst_idx, src_idx, acc_rows, 80, per_worker)

  aux, table2 = pl.pallas_call(
      _mid_kernel,
      grid=(grid,),
      in_specs=[
          pl.BlockSpec((ROW_BLOCK, 64), lambda i: (i, 0)),
          pl.BlockSpec((NC, ROW_BLOCK, 80), lambda i: (0, i, 0)),
          pl.BlockSpec((64, 3), lambda i: (0, 0)),
          pl.BlockSpec((64, 3), lambda i: (0, 0)),
          pl.BlockSpec((1, 3), lambda i: (0, 0)),
          pl.BlockSpec((1, 3), lambda i: (0, 0)),
      ],
      out_specs=[
          pl.BlockSpec((ROW_BLOCK, 8), lambda i: (i, 0)),
          pl.BlockSpec((ROW_BLOCK, 16), lambda i: (i, 0)),
      ],
      out_shape=[
          jax.ShapeDtypeStruct((V, 8), jnp.float32),
          jax.ShapeDtypeStruct((V, 16), jnp.float32),
      ],
  )(vw0, p1, w0_2.T, w1_2.T, b0_2[None, :], b1_2[None, :])

  p2 = _scatter_add_sc(table2, dst_idx, src_idx, acc_rows, 16, per_worker)

  out = pl.pallas_call(
      _final_kernel,
      grid=(grid,),
      in_specs=[
          pl.BlockSpec((ROW_BLOCK, 8), lambda i: (i, 0)),
          pl.BlockSpec((NC, ROW_BLOCK, 16), lambda i: (0, i, 0)),
      ],
      out_specs=pl.BlockSpec((ROW_BLOCK, 3), lambda i: (i, 0)),
      out_shape=jax.ShapeDtypeStruct((V, 3), jnp.float32),
  )(aux, p2)
  return out


# speedup vs baseline: 5.8069x; 5.8069x over previous
import functools

import jax
import jax.numpy as jnp
from jax import lax
from jax.experimental import pallas as pl
from jax.experimental.pallas import tpu as pltpu
from jax.experimental.pallas import tpu_sc as plsc

NC = 2
NS = 16
NW = NC * NS
CHUNK = 128
ROW_BLOCK = 2000


def _scatter_add_sc(table, dst_idx, src_idx, acc_rows, width, chunks_per_worker):
  stripe = acc_rows // NS

  mesh = plsc.VectorSubcoreMesh(core_axis_name="c", subcore_axis_name="s")

  @functools.partial(
      pl.kernel,
      out_type=jax.ShapeDtypeStruct((NC, acc_rows, width), jnp.float32),
      mesh=mesh,
      scratch_types=[
          pltpu.VMEM((CHUNK,), jnp.int32),
          pltpu.VMEM((CHUNK,), jnp.int32),
          pltpu.VMEM((CHUNK, width), jnp.float32),
          pltpu.VMEM((acc_rows // NS, width), jnp.float32),
          pltpu.VMEM_SHARED((acc_rows, width), jnp.float32),
          pltpu.SemaphoreType.DMA,
      ],
      compiler_params=pltpu.CompilerParams(use_tc_tiling_on_sc=False),
  )
  def k(table_hbm, dst_hbm, src_hbm, out_hbm, idx_d, idx_s, rows, zeros_v, acc, sem):
    cid = lax.axis_index("c")
    sid = lax.axis_index("s")
    wid = sid * NC + cid

    def zrow(r, carry):
      for c in range(width // 16):
        zeros_v[r, pl.ds(c * 16, 16)] = jnp.zeros((16,), jnp.float32)
      return carry
    lax.fori_loop(0, stripe, zrow, 0)
    pltpu.sync_copy(zeros_v, acc.at[pl.ds(sid * stripe, stripe)])
    plsc.subcore_barrier()

    def step(t, carry):
      base = (wid * chunks_per_worker + t) * CHUNK
      pltpu.sync_copy(dst_hbm.at[pl.ds(base, CHUNK)], idx_d)
      pltpu.sync_copy(src_hbm.at[pl.ds(base, CHUNK)], idx_s)
      pltpu.async_copy(table_hbm.at[idx_s], rows, sem).wait()
      pltpu.sync_copy(rows, acc.at[idx_d], add=True)
      return carry
    lax.fori_loop(0, chunks_per_worker, step, 0)

    plsc.subcore_barrier()
    pltpu.sync_copy(acc.at[pl.ds(sid * stripe, stripe)],
                    out_hbm.at[cid].at[pl.ds(sid * stripe, stripe)])

  return k(table, dst_idx, src_idx)


def _mm1_kernel(x_ref, w0t_ref, w1t_ref, b0_ref, b1_ref, vw0_ref, table_ref):
  x = x_ref[...]
  vw0 = jnp.dot(x, w0t_ref[...], preferred_element_type=jnp.float32)
  vw1 = jnp.dot(x, w1t_ref[...], preferred_element_type=jnp.float32)
  vw0_ref[...] = vw0 + b0_ref[...]
  r = x.shape[0]
  table_ref[...] = jnp.concatenate(
      [vw1 + b1_ref[...], jnp.ones((r, 1), jnp.float32),
       jnp.zeros((r, 15), jnp.float32)], axis=1)


def _mid_kernel(vw0_ref, p_ref, w0t_ref, w1t_ref, b0_ref, b1_ref, aux_ref,
                table_ref):
  p = p_ref[...]
  s = p[0] + p[1]
  nbr = s[:, :64]
  cnt = s[:, 64:65]
  dinv = 1.0 / cnt
  h = jnp.maximum((vw0_ref[...] + nbr) * dinv, 0.0)
  hw0 = jnp.dot(h, w0t_ref[...], preferred_element_type=jnp.float32) + b0_ref[...]
  hw1 = jnp.dot(h, w1t_ref[...], preferred_element_type=jnp.float32) + b1_ref[...]
  r = h.shape[0]
  aux_ref[...] = jnp.concatenate(
      [hw0, dinv, jnp.zeros((r, 4), jnp.float32)], axis=1)
  table_ref[...] = jnp.concatenate(
      [hw1, jnp.zeros((r, 13), jnp.float32)], axis=1)


def _final_kernel(aux_ref, p_ref, out_ref):
  p = p_ref[...]
  s = p[0] + p[1]
  out_ref[...] = (aux_ref[:, :3] + s[:, :3]) * aux_ref[:, 3:4]


def kernel(features, w0_1, b0_1, w1_1, b1_1, w0_2, b0_2, w1_2, b1_2, edges):
  V = features.shape[0]
  E = edges.shape[0]
  E2 = 2 * E
  per_worker = -(-E2 // (NW * CHUNK))
  pad_len = per_worker * NW * CHUNK - E2
  acc_rows = -(-(V + 1) // (NS * CHUNK)) * NS * CHUNK

  e0 = edges[:, 0]
  e1 = edges[:, 1]
  dst_idx = jnp.concatenate([e0, e1, jnp.full((pad_len,), V, jnp.int32)])
  src_idx = jnp.concatenate([e1, e0, jnp.zeros((pad_len,), jnp.int32)])

  grid = V // ROW_BLOCK

  vw0, table1 = pl.pallas_call(
      _mm1_kernel,
      grid=(grid,),
      in_specs=[
          pl.BlockSpec((ROW_BLOCK, 128), lambda i: (i, 0)),
          pl.BlockSpec((128, 64), lambda i: (0, 0)),
          pl.BlockSpec((128, 64), lambda i: (0, 0)),
          pl.BlockSpec((1, 64), lambda i: (0, 0)),
          pl.BlockSpec((1, 64), lambda i: (0, 0)),
      ],
      out_specs=[
          pl.BlockSpec((ROW_BLOCK, 64), lambda i: (i, 0)),
          pl.BlockSpec((ROW_BLOCK, 80), lambda i: (i, 0)),
      ],
      out_shape=[
          jax.ShapeDtypeStruct((V, 64), jnp.float32),
          jax.ShapeDtypeStruct((V, 80), jnp.float32),
      ],
  )(features, w0_1.T, w1_1.T, b0_1[None, :], b1_1[None, :])

  p1 = _scatter_add_sc(table1, dst_idx, src_idx, acc_rows, 80, per_worker)

  aux, table2 = pl.pallas_call(
      _mid_kernel,
      grid=(grid,),
      in_specs=[
          pl.BlockSpec((ROW_BLOCK, 64), lambda i: (i, 0)),
          pl.BlockSpec((NC, ROW_BLOCK, 80), lambda i: (0, i, 0)),
          pl.BlockSpec((64, 3), lambda i: (0, 0)),
          pl.BlockSpec((64, 3), lambda i: (0, 0)),
          pl.BlockSpec((1, 3), lambda i: (0, 0)),
          pl.BlockSpec((1, 3), lambda i: (0, 0)),
      ],
      out_specs=[
          pl.BlockSpec((ROW_BLOCK, 8), lambda i: (i, 0)),
          pl.BlockSpec((ROW_BLOCK, 16), lambda i: (i, 0)),
      ],
      out_shape=[
          jax.ShapeDtypeStruct((V, 8), jnp.float32),
          jax.ShapeDtypeStruct((V, 16), jnp.float32),
      ],
  )(vw0, p1, w0_2.T, w1_2.T, b0_2[None, :], b1_2[None, :])

  p2 = _scatter_add_sc(table2, dst_idx, src_idx, acc_rows, 16, per_worker)

  out = pl.pallas_call(
      _final_kernel,
      grid=(grid,),
      in_specs=[
          pl.BlockSpec((ROW_BLOCK, 8), lambda i: (i, 0)),
          pl.BlockSpec((NC, ROW_BLOCK, 16), lambda i: (0, i, 0)),
      ],
      out_specs=pl.BlockSpec((ROW_BLOCK, 3), lambda i: (i, 0)),
      out_shape=jax.ShapeDtypeStruct((V, 3), jnp.float32),
  )(aux, p2)
  return out


# speedup vs baseline: 9.6223x; 1.6570x over previous
import functools

import jax
import jax.numpy as jnp
from jax import lax
from jax.experimental import pallas as pl
from jax.experimental.pallas import tpu as pltpu
from jax.experimental.pallas import tpu_sc as plsc

NC = 2
NS = 16
NW = NC * NS
CHUNK = 128
ROW_BLOCK = 2000


def _scatter_add_sc(table, dst_idx, src_idx, acc_rows, width, chunks_per_worker):
  stripe = acc_rows // NS
  T = chunks_per_worker

  mesh = plsc.VectorSubcoreMesh(core_axis_name="c", subcore_axis_name="s")

  @functools.partial(
      pl.kernel,
      out_type=jax.ShapeDtypeStruct((NC, acc_rows, width), jnp.float32),
      mesh=mesh,
      scratch_types=[
          pltpu.VMEM((T, CHUNK), jnp.int32),
          pltpu.VMEM((T, CHUNK), jnp.int32),
          pltpu.VMEM((CHUNK, width), jnp.float32),
          pltpu.VMEM((CHUNK, width), jnp.float32),
          pltpu.VMEM((CHUNK, width), jnp.float32),
          pltpu.VMEM_SHARED((acc_rows, width), jnp.float32),
          pltpu.SemaphoreType.DMA,
          pltpu.SemaphoreType.DMA,
          pltpu.SemaphoreType.DMA,
      ],
      compiler_params=pltpu.CompilerParams(use_tc_tiling_on_sc=False),
  )
  def k(table_hbm, dst_hbm, src_hbm, out_hbm, idx_d, idx_s, rows0, rows1,
        zeros_v, acc, sem0, sem1, semi):
    cid = lax.axis_index("c")
    sid = lax.axis_index("s")
    wid = sid * NC + cid

    cp_d = pltpu.async_copy(dst_hbm.at[wid], idx_d, semi)
    cp_s = pltpu.async_copy(src_hbm.at[wid], idx_s, semi)

    def zrow(r, carry):
      for c in range(width // 16):
        zeros_v[r, pl.ds(c * 16, 16)] = jnp.zeros((16,), jnp.float32)
      return carry
    lax.fori_loop(0, CHUNK, zrow, 0)
    for b in range(stripe // CHUNK):
      pltpu.sync_copy(zeros_v, acc.at[pl.ds(sid * stripe + b * CHUNK, CHUNK)])
    cp_d.wait()
    cp_s.wait()
    plsc.subcore_barrier()

    def issue(t, buf, sem):
      pltpu.async_copy(table_hbm.at[idx_s.at[t]], buf, sem)

    def drain(buf, sem):
      pltpu.make_async_copy(table_hbm.at[idx_s.at[0]], buf, sem).wait()

    issue(0, rows0, sem0)
    issue(1, rows1, sem1)

    def body(tt, carry):
      t0 = tt * 2
      drain(rows0, sem0)
      pltpu.sync_copy(rows0, acc.at[idx_d.at[t0]], add=True)
      issue(t0 + 2, rows0, sem0)
      drain(rows1, sem1)
      pltpu.sync_copy(rows1, acc.at[idx_d.at[t0 + 1]], add=True)
      issue(t0 + 3, rows1, sem1)
      return carry
    lax.fori_loop(0, T // 2 - 1, body, 0)

    drain(rows0, sem0)
    pltpu.sync_copy(rows0, acc.at[idx_d.at[T - 2]], add=True)
    drain(rows1, sem1)
    pltpu.sync_copy(rows1, acc.at[idx_d.at[T - 1]], add=True)

    plsc.subcore_barrier()
    pltpu.sync_copy(acc.at[pl.ds(sid * stripe, stripe)],
                    out_hbm.at[cid].at[pl.ds(sid * stripe, stripe)])

  return k(table, dst_idx, src_idx)


def _mm1_kernel(x_ref, w0t_ref, w1t_ref, b0_ref, b1_ref, vw0_ref, table_ref):
  x = x_ref[...]
  vw0 = jnp.dot(x, w0t_ref[...], preferred_element_type=jnp.float32)
  vw1 = jnp.dot(x, w1t_ref[...], preferred_element_type=jnp.float32)
  vw0_ref[...] = vw0 + b0_ref[...]
  r = x.shape[0]
  table_ref[...] = jnp.concatenate(
      [vw1 + b1_ref[...], jnp.ones((r, 1), jnp.float32),
       jnp.zeros((r, 15), jnp.float32)], axis=1)


def _mid_kernel(vw0_ref, p_ref, w0t_ref, w1t_ref, b0_ref, b1_ref, aux_ref,
                table_ref):
  p = p_ref[...]
  s = p[0] + p[1]
  nbr = s[:, :64]
  cnt = s[:, 64:65]
  dinv = 1.0 / cnt
  h = jnp.maximum((vw0_ref[...] + nbr) * dinv, 0.0)
  hw0 = jnp.dot(h, w0t_ref[...], preferred_element_type=jnp.float32) + b0_ref[...]
  hw1 = jnp.dot(h, w1t_ref[...], preferred_element_type=jnp.float32) + b1_ref[...]
  r = h.shape[0]
  aux_ref[...] = jnp.concatenate(
      [hw0, dinv, jnp.zeros((r, 4), jnp.float32)], axis=1)
  table_ref[...] = jnp.concatenate(
      [hw1, jnp.zeros((r, 13), jnp.float32)], axis=1)


def _final_kernel(aux_ref, p_ref, out_ref):
  p = p_ref[...]
  s = p[0] + p[1]
  out_ref[...] = (aux_ref[:, :3] + s[:, :3]) * aux_ref[:, 3:4]


def kernel(features, w0_1, b0_1, w1_1, b1_1, w0_2, b0_2, w1_2, b1_2, edges):
  V = features.shape[0]
  E = edges.shape[0]
  E2 = 2 * E
  per_worker = -(-E2 // (NW * CHUNK))
  per_worker += per_worker % 2
  pad_len = per_worker * NW * CHUNK - E2
  acc_rows = -(-(V + 1) // (NS * CHUNK)) * NS * CHUNK

  e0 = edges[:, 0]
  e1 = edges[:, 1]
  dst_idx = jnp.concatenate(
      [e0, e1, jnp.full((pad_len,), V, jnp.int32)]).reshape(
          NW, per_worker, CHUNK)
  src_idx = jnp.concatenate(
      [e1, e0, jnp.zeros((pad_len,), jnp.int32)]).reshape(
          NW, per_worker, CHUNK)

  grid = V // ROW_BLOCK

  vw0, table1 = pl.pallas_call(
      _mm1_kernel,
      grid=(grid,),
      in_specs=[
          pl.BlockSpec((ROW_BLOCK, 128), lambda i: (i, 0)),
          pl.BlockSpec((128, 64), lambda i: (0, 0)),
          pl.BlockSpec((128, 64), lambda i: (0, 0)),
          pl.BlockSpec((1, 64), lambda i: (0, 0)),
          pl.BlockSpec((1, 64), lambda i: (0, 0)),
      ],
      out_specs=[
          pl.BlockSpec((ROW_BLOCK, 64), lambda i: (i, 0)),
          pl.BlockSpec((ROW_BLOCK, 80), lambda i: (i, 0)),
      ],
      out_shape=[
          jax.ShapeDtypeStruct((V, 64), jnp.float32),
          jax.ShapeDtypeStruct((V, 80), jnp.float32),
      ],
  )(features, w0_1.T, w1_1.T, b0_1[None, :], b1_1[None, :])

  p1 = _scatter_add_sc(table1, dst_idx, src_idx, acc_rows, 80, per_worker)

  aux, table2 = pl.pallas_call(
      _mid_kernel,
      grid=(grid,),
      in_specs=[
          pl.BlockSpec((ROW_BLOCK, 64), lambda i: (i, 0)),
          pl.BlockSpec((NC, ROW_BLOCK, 80), lambda i: (0, i, 0)),
          pl.BlockSpec((64, 3), lambda i: (0, 0)),
          pl.BlockSpec((64, 3), lambda i: (0, 0)),
          pl.BlockSpec((1, 3), lambda i: (0, 0)),
          pl.BlockSpec((1, 3), lambda i: (0, 0)),
      ],
      out_specs=[
          pl.BlockSpec((ROW_BLOCK, 8), lambda i: (i, 0)),
          pl.BlockSpec((ROW_BLOCK, 16), lambda i: (i, 0)),
      ],
      out_shape=[
          jax.ShapeDtypeStruct((V, 8), jnp.float32),
          jax.ShapeDtypeStruct((V, 16), jnp.float32),
      ],
  )(vw0, p1, w0_2.T, w1_2.T, b0_2[None, :], b1_2[None, :])

  p2 = _scatter_add_sc(table2, dst_idx, src_idx, acc_rows, 16, per_worker)

  out = pl.pallas_call(
      _final_kernel,
      grid=(grid,),
      in_specs=[
          pl.BlockSpec((ROW_BLOCK, 8), lambda i: (i, 0)),
          pl.BlockSpec((NC, ROW_BLOCK, 16), lambda i: (0, i, 0)),
      ],
      out_specs=pl.BlockSpec((ROW_BLOCK, 3), lambda i: (i, 0)),
      out_shape=jax.ShapeDtypeStruct((V, 3), jnp.float32),
  )(aux, p2)
  return out


# speedup vs baseline: 10.0062x; 1.0399x over previous
import functools

import jax
import jax.numpy as jnp
from jax import lax
from jax.experimental import pallas as pl
from jax.experimental.pallas import tpu as pltpu
from jax.experimental.pallas import tpu_sc as plsc

NC = 2
NS = 16
NW = NC * NS
CHUNK = 128
NBUF = 2
CW = 16
ROW_BLOCK = 2000


def _scatter_add_sc(table, dst_idx, src_idx, acc_rows, width,
                    chunks_per_worker, with_counts):
  stripe = acc_rows // NS
  T = chunks_per_worker

  mesh = plsc.VectorSubcoreMesh(core_axis_name="c", subcore_axis_name="s")

  out_type = [jax.ShapeDtypeStruct((NC, acc_rows, width), jnp.float32)]
  scratch = [
      pltpu.VMEM((T, CHUNK), jnp.int32),
      pltpu.VMEM((T, CHUNK), jnp.int32),
  ] + [pltpu.VMEM((CHUNK, width), jnp.float32) for _ in range(NBUF)] + [
      pltpu.VMEM((CHUNK, width), jnp.float32),
      pltpu.VMEM_SHARED((acc_rows, width), jnp.float32),
  ] + [pltpu.SemaphoreType.DMA for _ in range(NBUF + 1)]
  if with_counts:
    out_type.append(jax.ShapeDtypeStruct((NC, acc_rows, CW), jnp.float32))
    scratch += [
        pltpu.VMEM((CHUNK, CW), jnp.float32),
        pltpu.VMEM_SHARED((acc_rows, CW), jnp.float32),
        pltpu.VMEM((CHUNK, CW), jnp.float32),
    ]

  @functools.partial(
      pl.kernel,
      out_type=tuple(out_type) if with_counts else out_type[0],
      mesh=mesh,
      scratch_types=scratch,
      compiler_params=pltpu.CompilerParams(use_tc_tiling_on_sc=False),
  )
  def k(table_hbm, dst_hbm, src_hbm, *rest):
    if with_counts:
      (out_hbm, cout_hbm, idx_d, idx_s, *bufs) = rest
      rows = bufs[:NBUF]
      zeros_v = bufs[NBUF]
      acc = bufs[NBUF + 1]
      sems = bufs[NBUF + 2:2 * NBUF + 3]
      ones_v, cacc, czero_v = bufs[2 * NBUF + 3:]
    else:
      (out_hbm, idx_d, idx_s, *bufs) = rest
      rows = bufs[:NBUF]
      zeros_v = bufs[NBUF]
      acc = bufs[NBUF + 1]
      sems = bufs[NBUF + 2:2 * NBUF + 3]
    semi = sems[NBUF]
    cid = lax.axis_index("c")
    sid = lax.axis_index("s")
    wid = sid * NC + cid

    cp_d = pltpu.async_copy(dst_hbm.at[wid], idx_d, semi)
    cp_s = pltpu.async_copy(src_hbm.at[wid], idx_s, semi)

    def zrow(r, carry):
      for c in range(width // 16):
        zeros_v[r, pl.ds(c * 16, 16)] = jnp.zeros((16,), jnp.float32)
      if with_counts:
        ones_v[r, :] = jnp.ones((CW,), jnp.float32)
        czero_v[r, :] = jnp.zeros((CW,), jnp.float32)
      return carry
    lax.fori_loop(0, CHUNK, zrow, 0)
    for b in range(stripe // CHUNK):
      pltpu.sync_copy(zeros_v, acc.at[pl.ds(sid * stripe + b * CHUNK, CHUNK)])
      if with_counts:
        pltpu.sync_copy(czero_v,
                        cacc.at[pl.ds(sid * stripe + b * CHUNK, CHUNK)])
    cp_d.wait()
    cp_s.wait()
    plsc.subcore_barrier()

    def issue(t, buf, sem):
      pltpu.async_copy(table_hbm.at[idx_s.at[t]], buf, sem)

    def drain(buf, sem):
      pltpu.make_async_copy(table_hbm.at[idx_s.at[0]], buf, sem).wait()

    def scatter(t, buf):
      pltpu.sync_copy(buf, acc.at[idx_d.at[t]], add=True)
      if with_counts:
        pltpu.sync_copy(ones_v, cacc.at[idx_d.at[t]], add=True)

    for b in range(NBUF):
      issue(b, rows[b], sems[b])

    def body(g, carry):
      t0 = g * NBUF
      for b in range(NBUF):
        drain(rows[b], sems[b])
        scatter(t0 + b, rows[b])
        issue(t0 + NBUF + b, rows[b], sems[b])
      return carry
    lax.fori_loop(0, T // NBUF - 1, body, 0)

    for b in range(NBUF):
      drain(rows[b], sems[b])
      scatter(T - NBUF + b, rows[b])

    plsc.subcore_barrier()
    pltpu.sync_copy(acc.at[pl.ds(sid * stripe, stripe)],
                    out_hbm.at[cid].at[pl.ds(sid * stripe, stripe)])
    if with_counts:
      pltpu.sync_copy(cacc.at[pl.ds(sid * stripe, stripe)],
                      cout_hbm.at[cid].at[pl.ds(sid * stripe, stripe)])

  return k(table, dst_idx, src_idx)


def _mm1_kernel(x_ref, w0t_ref, w1t_ref, b0_ref, b1_ref, vw0_ref, table_ref):
  x = x_ref[...]
  vw0 = jnp.dot(x, w0t_ref[...], preferred_element_type=jnp.float32)
  vw1 = jnp.dot(x, w1t_ref[...], preferred_element_type=jnp.float32)
  vw0_ref[...] = vw0 + b0_ref[...]
  table_ref[...] = vw1 + b1_ref[...]


def _mid_kernel(vw0_ref, p_ref, c_ref, w0t_ref, w1t_ref, b0_ref, b1_ref,
                aux_ref, table_ref):
  p = p_ref[...]
  nbr = p[0] + p[1]
  c = c_ref[...]
  cnt = (c[0] + c[1])[:, 0:1]
  dinv = 1.0 / cnt
  h = jnp.maximum((vw0_ref[...] + nbr) * dinv, 0.0)
  hw0 = jnp.dot(h, w0t_ref[...], preferred_element_type=jnp.float32) + b0_ref[...]
  hw1 = jnp.dot(h, w1t_ref[...], preferred_element_type=jnp.float32) + b1_ref[...]
  r = h.shape[0]
  aux_ref[...] = jnp.concatenate(
      [hw0, dinv, jnp.zeros((r, 4), jnp.float32)], axis=1)
  table_ref[...] = jnp.concatenate(
      [hw1, jnp.zeros((r, 13), jnp.float32)], axis=1)


def _final_kernel(aux_ref, p_ref, out_ref):
  p = p_ref[...]
  s = p[0] + p[1]
  out_ref[...] = (aux_ref[:, :3] + s[:, :3]) * aux_ref[:, 3:4]


def kernel(features, w0_1, b0_1, w1_1, b1_1, w0_2, b0_2, w1_2, b1_2, edges):
  V = features.shape[0]
  E = edges.shape[0]
  E2 = 2 * E
  per_worker = -(-E2 // (NW * CHUNK))
  per_worker += (-per_worker) % NBUF
  pad_len = per_worker * NW * CHUNK - E2
  acc_rows = -(-(V + 1) // (NS * CHUNK)) * NS * CHUNK

  e0 = edges[:, 0]
  e1 = edges[:, 1]
  dst_idx = jnp.concatenate(
      [e0, e1, jnp.full((pad_len,), V, jnp.int32)]).reshape(
          NW, per_worker, CHUNK)
  src_idx = jnp.concatenate(
      [e1, e0, jnp.zeros((pad_len,), jnp.int32)]).reshape(
          NW, per_worker, CHUNK)

  grid = V // ROW_BLOCK

  vw0, table1 = pl.pallas_call(
      _mm1_kernel,
      grid=(grid,),
      in_specs=[
          pl.BlockSpec((ROW_BLOCK, 128), lambda i: (i, 0)),
          pl.BlockSpec((128, 64), lambda i: (0, 0)),
          pl.BlockSpec((128, 64), lambda i: (0, 0)),
          pl.BlockSpec((1, 64), lambda i: (0, 0)),
          pl.BlockSpec((1, 64), lambda i: (0, 0)),
      ],
      out_specs=[
          pl.BlockSpec((ROW_BLOCK, 64), lambda i: (i, 0)),
          pl.BlockSpec((ROW_BLOCK, 64), lambda i: (i, 0)),
      ],
      out_shape=[
          jax.ShapeDtypeStruct((V, 64), jnp.float32),
          jax.ShapeDtypeStruct((V, 64), jnp.float32),
      ],
  )(features, w0_1.T, w1_1.T, b0_1[None, :], b1_1[None, :])

  p1, c1 = _scatter_add_sc(table1, dst_idx, src_idx, acc_rows, 64,
                           per_worker, True)

  aux, table2 = pl.pallas_call(
      _mid_kernel,
      grid=(grid,),
      in_specs=[
          pl.BlockSpec((ROW_BLOCK, 64), lambda i: (i, 0)),
          pl.BlockSpec((NC, ROW_BLOCK, 64), lambda i: (0, i, 0)),
          pl.BlockSpec((NC, ROW_BLOCK, CW), lambda i: (0, i, 0)),
          pl.BlockSpec((64, 3), lambda i: (0, 0)),
          pl.BlockSpec((64, 3), lambda i: (0, 0)),
          pl.BlockSpec((1, 3), lambda i: (0, 0)),
          pl.BlockSpec((1, 3), lambda i: (0, 0)),
      ],
      out_specs=[
          pl.BlockSpec((ROW_BLOCK, 8), lambda i: (i, 0)),
          pl.BlockSpec((ROW_BLOCK, 16), lambda i: (i, 0)),
      ],
      out_shape=[
          jax.ShapeDtypeStruct((V, 8), jnp.float32),
          jax.ShapeDtypeStruct((V, 16), jnp.float32),
      ],
  )(vw0, p1, c1, w0_2.T, w1_2.T, b0_2[None, :], b1_2[None, :])

  p2 = _scatter_add_sc(table2, dst_idx, src_idx, acc_rows, 16,
                       per_worker, False)

  out = pl.pallas_call(
      _final_kernel,
      grid=(grid,),
      in_specs=[
          pl.BlockSpec((ROW_BLOCK, 8), lambda i: (i, 0)),
          pl.BlockSpec((NC, ROW_BLOCK, 16), lambda i: (0, i, 0)),
      ],
      out_specs=pl.BlockSpec((ROW_BLOCK, 3), lambda i: (i, 0)),
      out_shape=jax.ShapeDtypeStruct((V, 3), jnp.float32),
  )(aux, p2)
  return out
